# baseline (speedup 1.0000x reference)
import functools
import jax
import jax.numpy as jnp
from jax import lax
from jax.experimental import pallas as pl
from jax.experimental.pallas import tpu as pltpu
from jax.experimental.pallas import tpu_sc as plsc

R = 5
DIN = 128
D = 64
C = 96
NCH = 132
EPAD = 202752
RNG = 12512
ACCR = 12544
NT = 16


def _transform_body(x_ref, wrev_ref, brev_ref, wagg_ref, out_ref):
    x = x_ref[...]
    blk = x.shape[0]
    for r in range(R):
        m = jnp.dot(x, wrev_ref[r], preferred_element_type=jnp.float32) + brev_ref[r]
        p = jnp.dot(m, wagg_ref[pl.ds(D * (r + 1), D), :],
                    preferred_element_type=jnp.float32)
        out_ref[r] = jnp.concatenate(
            [p, jnp.ones((blk, 1), jnp.float32),
             jnp.zeros((blk, DIN - D - 1), jnp.float32)], axis=1)


def _item_transform(x, wrev, brev, wagg, block=2000):
    n = x.shape[0]
    return pl.pallas_call(
        _transform_body,
        grid=(n // block,),
        in_specs=[
            pl.BlockSpec((block, DIN), lambda i: (i, 0)),
            pl.BlockSpec((R, DIN, D), lambda i: (0, 0, 0)),
            pl.BlockSpec((R, D), lambda i: (0, 0)),
            pl.BlockSpec((D * (R + 1), D), lambda i: (0, 0)),
        ],
        out_specs=pl.BlockSpec((R, block, DIN), lambda i: (0, i, 0)),
        out_shape=jax.ShapeDtypeStruct((R, n, DIN), jnp.float32),
    )(x, wrev, brev, wagg)


def _final_body(uf_ref, s_ref, wagg_ref, bagg_ref, out_ref):
    acc = jnp.dot(uf_ref[...], wagg_ref[pl.ds(0, D), :],
                  preferred_element_type=jnp.float32)
    lanes = lax.broadcasted_iota(jnp.int32, (1, DIN), 1)
    cnt_hot = jnp.where(lanes == D, 1.0, 0.0)
    for r in range(R):
        srow = s_ref[r]
        cnt = jnp.sum(srow * cnt_hot, axis=1, keepdims=True)
        inv = 1.0 / jnp.maximum(cnt, 1.0)
        acc = acc + srow[:, :D] * inv
    acc = acc + bagg_ref[...]
    out_ref[...] = jnp.where(acc >= 0, acc, 0.01 * acc)


def _final(uf, s, wagg, bagg, block=2000):
    u = uf.shape[0]
    return pl.pallas_call(
        _final_body,
        grid=(u // block,),
        in_specs=[
            pl.BlockSpec((block, D), lambda i: (i, 0)),
            pl.BlockSpec((R, block, DIN), lambda i: (0, i, 0)),
            pl.BlockSpec((D * (R + 1), D), lambda i: (0, 0)),
            pl.BlockSpec((1, D), lambda i: (0, 0)),
        ],
        out_specs=pl.BlockSpec((block, D), lambda i: (i, 0)),
        out_shape=jax.ShapeDtypeStruct((u, D), jnp.float32),
    )(uf, s, wagg, bagg)


SOUT = 8 * RNG


def _edge_stage(p, src1, dst1, n_users):
    nitems = p.shape[1]
    p = p.reshape(R * nitems, DIN)
    mesh = plsc.VectorSubcoreMesh(core_axis_name="c", subcore_axis_name="s")

    @functools.partial(
        pl.kernel, mesh=mesh,
        out_type=jax.ShapeDtypeStruct((R, SOUT, DIN), jnp.float32),
        scratch_types=[
            pltpu.VMEM_SHARED((ACCR, DIN), jnp.float32),
            pltpu.VMEM((C,), jnp.int32), pltpu.VMEM((C,), jnp.int32),
            pltpu.VMEM((C,), jnp.int32), pltpu.VMEM((C,), jnp.int32),
            pltpu.VMEM((C,), jnp.int32), pltpu.VMEM((C,), jnp.int32),
            pltpu.VMEM((C, DIN), jnp.float32),
            pltpu.VMEM((C, DIN), jnp.float32),
            pltpu.SemaphoreType.DMA, pltpu.SemaphoreType.DMA,
            pltpu.SemaphoreType.DMA, pltpu.SemaphoreType.DMA,
            pltpu.SemaphoreType.DMA, pltpu.SemaphoreType.DMA,
        ],
    )
    def k_sc(p_hbm, src_hbm, dst_hbm, s_hbm, acc,
             src_v0, src_v1, dst_v0, dst_v1, dstl_v0, dstl_v1,
             rows_v0, rows_v1,
             sem_i0, sem_i1, sem_g0, sem_g1, sem_s0, sem_s1):
        c = lax.axis_index("c")
        s = lax.axis_index("s")
        zvec = jnp.zeros((16,), jnp.float32)
        srcs, dsts = (src_v0, src_v1), (dst_v0, dst_v1)
        dstls, rows = (dstl_v0, dstl_v1), (rows_v0, rows_v1)
        semi, semg, sems = (sem_i0, sem_i1), (sem_g0, sem_g1), (sem_s0, sem_s1)
        row0 = s * 784

        def zero_acc():
            def zfill(i, carry):
                for k in range(DIN // 16):
                    rows_v0[i, pl.ds(k * 16, 16)] = zvec
                return carry
            lax.fori_loop(0, C, zfill, 0)
            for k in range(8):
                pltpu.sync_copy(rows_v0, acc.at[pl.ds(row0 + C * k, C)])
            pltpu.sync_copy(rows_v0.at[pl.ds(0, 16)],
                            acc.at[pl.ds(row0 + 8 * C, 16)])

        def issue_idx(b, r, i):
            off = pl.multiple_of(r * EPAD + (s + NT * i) * C, 8)
            pltpu.async_copy(src_hbm.at[pl.ds(off, C)], srcs[b], semi[b])
            pltpu.async_copy(dst_hbm.at[pl.ds(off, C)], dsts[b], semi[b])

        def wait_idx(b):
            pltpu.make_async_copy(src_hbm.at[pl.ds(0, C)], srcs[b],
                                  semi[b]).wait()
            pltpu.make_async_copy(src_hbm.at[pl.ds(0, C)], dsts[b],
                                  semi[b]).wait()

        def start_gather(b):
            pltpu.async_copy(p_hbm.at[srcs[b]], rows[b], semg[b])

        def wait_gather(b):
            pltpu.make_async_copy(p_hbm.at[srcs[b]], rows[b], semg[b]).wait()

        def start_scatter(b):
            pltpu.async_copy(rows[b], acc.at[dstls[b]], sems[b], add=True)

        def wait_scatter(b):
            pltpu.make_async_copy(rows[b], acc.at[dstls[b]], sems[b]).wait()

        def remap(b, base, rbase):
            def body(k, carry):
                sl = pl.ds(k * 16, 16)
                loc = dsts[b][sl] - base
                ok = (loc >= 0) & (loc < RNG)
                dstls[b][sl] = jnp.where(ok, loc, RNG)
                srcs[b][sl] = srcs[b][sl] + rbase
                return carry
            lax.fori_loop(0, C // 16, body, 0)

        def edge_pass(r, base, rbase):
            issue_idx(0, r, 0)
            issue_idx(1, r, 1)
            wait_idx(0)
            remap(0, base, rbase)
            start_gather(0)

            def slot(i, b, nb):
                @pl.when(i < NCH - 1)
                def _():
                    wait_idx(nb)

                    @pl.when(i >= 1)
                    def _():
                        wait_scatter(nb)
                    remap(nb, base, rbase)
                    start_gather(nb)
                wait_gather(b)

                @pl.when(i < NCH - 2)
                def _():
                    issue_idx(b, r, i + 2)
                start_scatter(b)

            def body(i2, carry):
                slot(2 * i2, 0, 1)
                slot(2 * i2 + 1, 1, 0)
                return carry
            lax.fori_loop(0, NCH // 2, body, 0)
            wait_scatter(0)
            wait_scatter(1)

        def drain(r, u0):
            @pl.when(s < 15)
            def _():
                pltpu.sync_copy(acc.at[pl.ds(row0, 784)],
                                s_hbm.at[r, pl.ds(u0 + row0, 784)])

            @pl.when(s == 15)
            def _():
                pltpu.sync_copy(acc.at[pl.ds(11760, 752)],
                                s_hbm.at[r, pl.ds(u0 + 11760, 752)])

        for gg in range(4):
            base = (c * 4 + gg) * RNG
            for r in range(R):
                zero_acc()
                plsc.subcore_barrier()
                edge_pass(r, base, r * nitems)
                plsc.subcore_barrier()
                drain(r, base)

    return k_sc(p, src1, dst1)


def kernel(item_features, user_features, item_nids,
           edge_src_0, edge_dst_0, edge_src_1, edge_dst_1,
           edge_src_2, edge_dst_2, edge_src_3, edge_dst_3,
           edge_src_4, edge_dst_4,
           item_id_table,
           W_rev_0, b_rev_0, W_rev_1, b_rev_1, W_rev_2, b_rev_2,
           W_rev_3, b_rev_3, W_rev_4, b_rev_4,
           W_agg, b_agg):
    n_users = user_features.shape[0]
    e = edge_src_0.shape[0]
    item_id_emb = jnp.take(item_id_table, item_nids, axis=0)
    x = jnp.concatenate([item_features, item_id_emb], axis=1)
    wrev = jnp.stack([W_rev_0, W_rev_1, W_rev_2, W_rev_3, W_rev_4])
    brev = jnp.stack([b_rev_0, b_rev_1, b_rev_2, b_rev_3, b_rev_4])

    p = _item_transform(x, wrev, brev, W_agg)

    spad = jnp.zeros((EPAD - e,), jnp.int32)
    dpad = jnp.full((EPAD - e,), 1 << 20, jnp.int32)
    src1 = jnp.concatenate([
        edge_src_0.astype(jnp.int32), spad, edge_src_1.astype(jnp.int32), spad,
        edge_src_2.astype(jnp.int32), spad, edge_src_3.astype(jnp.int32), spad,
        edge_src_4.astype(jnp.int32), spad])
    dst1 = jnp.concatenate([
        edge_dst_0.astype(jnp.int32), dpad, edge_dst_1.astype(jnp.int32), dpad,
        edge_dst_2.astype(jnp.int32), dpad, edge_dst_3.astype(jnp.int32), dpad,
        edge_dst_4.astype(jnp.int32), dpad])

    s = _edge_stage(p, src1, dst1, n_users)

    return _final(user_features, s, W_agg, b_agg.reshape(1, D))


# speedup vs baseline: 1.3431x; 1.3431x over previous
import functools
import jax
import jax.numpy as jnp
from jax import lax
from jax.experimental import pallas as pl
from jax.experimental.pallas import tpu as pltpu
from jax.experimental.pallas import tpu_sc as plsc

R = 5
DIN = 128
D = 64
C = 256
EPAD = 200192
RNG = 10016
ACCR = 10112
NT = 16


def _transform_body(x_ref, wrev_ref, brev_ref, wagg_ref, out_ref):
    x = x_ref[...]
    blk = x.shape[0]
    for r in range(R):
        m = jnp.dot(x, wrev_ref[r], preferred_element_type=jnp.float32) + brev_ref[r]
        p = jnp.dot(m, wagg_ref[pl.ds(D * (r + 1), D), :],
                    preferred_element_type=jnp.float32)
        out_ref[r] = jnp.concatenate(
            [p, jnp.ones((blk, 1), jnp.float32),
             jnp.zeros((blk, DIN - D - 1), jnp.float32)], axis=1)


def _item_transform(x, wrev, brev, wagg, block=2000):
    n = x.shape[0]
    return pl.pallas_call(
        _transform_body,
        grid=(n // block,),
        in_specs=[
            pl.BlockSpec((block, DIN), lambda i: (i, 0)),
            pl.BlockSpec((R, DIN, D), lambda i: (0, 0, 0)),
            pl.BlockSpec((R, D), lambda i: (0, 0)),
            pl.BlockSpec((D * (R + 1), D), lambda i: (0, 0)),
        ],
        out_specs=pl.BlockSpec((R, block, DIN), lambda i: (0, i, 0)),
        out_shape=jax.ShapeDtypeStruct((R, n, DIN), jnp.float32),
    )(x, wrev, brev, wagg)


def _final_body(uf_ref, s_ref, wagg_ref, bagg_ref, out_ref):
    acc = jnp.dot(uf_ref[...], wagg_ref[pl.ds(0, D), :],
                  preferred_element_type=jnp.float32)
    lanes = lax.broadcasted_iota(jnp.int32, (1, DIN), 1)
    cnt_hot = jnp.where(lanes == D, 1.0, 0.0)
    for r in range(R):
        srow = s_ref[r]
        cnt = jnp.sum(srow * cnt_hot, axis=1, keepdims=True)
        inv = 1.0 / jnp.maximum(cnt, 1.0)
        acc = acc + srow[:, :D] * inv
    acc = acc + bagg_ref[...]
    out_ref[...] = jnp.where(acc >= 0, acc, 0.01 * acc)


def _final(uf, s, wagg, bagg, block=2000):
    u = uf.shape[0]
    return pl.pallas_call(
        _final_body,
        grid=(u // block,),
        in_specs=[
            pl.BlockSpec((block, D), lambda i: (i, 0)),
            pl.BlockSpec((R, block, DIN), lambda i: (0, i, 0)),
            pl.BlockSpec((D * (R + 1), D), lambda i: (0, 0)),
            pl.BlockSpec((1, D), lambda i: (0, 0)),
        ],
        out_specs=pl.BlockSpec((block, D), lambda i: (i, 0)),
        out_shape=jax.ShapeDtypeStruct((u, D), jnp.float32),
    )(uf, s, wagg, bagg)


SOUT = 10 * RNG


def _edge_stage(p, sd, n_users):
    nitems = p.shape[1]
    p = p.reshape(R * nitems, DIN)
    nchunks = EPAD // C
    iters_long = nchunks - NT * (nchunks // NT)
    mesh = plsc.VectorSubcoreMesh(core_axis_name="c", subcore_axis_name="s")

    @functools.partial(
        pl.kernel, mesh=mesh,
        out_type=jax.ShapeDtypeStruct((R, SOUT, DIN), jnp.float32),
        scratch_types=[
            pltpu.VMEM_SHARED((ACCR, DIN), jnp.float32),
            pltpu.VMEM((2 * C,), jnp.int32),
            pltpu.VMEM((2, 128), jnp.int32),
            pltpu.VMEM((C, DIN), jnp.float32),
            pltpu.SemaphoreType.DMA,
        ],
    )
    def k_sc(p_hbm, sd_hbm, s_hbm, acc, sd_v, dstl_v, rows_v, sem):
        c = lax.axis_index("c")
        s = lax.axis_index("s")
        zvec = jnp.zeros((16,), jnp.float32)
        nhi = jnp.where(s < iters_long, nchunks // NT + 1, nchunks // NT)
        row0 = s * 632

        def zero_acc():
            def zfill(i, carry):
                for k in range(DIN // 16):
                    rows_v[i, pl.ds(k * 16, 16)] = zvec
                return carry
            lax.fori_loop(0, C, zfill, 0)
            pltpu.sync_copy(rows_v, acc.at[pl.ds(row0, C)])
            pltpu.sync_copy(rows_v.at[pl.ds(0, 256)],
                            acc.at[pl.ds(row0 + C, 256)])
            pltpu.sync_copy(rows_v.at[pl.ds(0, 120)],
                            acc.at[pl.ds(row0 + C + 256, 120)])

        def remap(base, rbase):
            def body(k, carry):
                sl = pl.ds(k * 16, 16)
                loc = sd_v[pl.ds(C + k * 16, 16)] - base
                ok = (loc >= 0) & (loc < RNG)
                dstl_v[k // 8, pl.ds((k % 8) * 16, 16)] = (
                    jnp.where(ok, loc, RNG))
                sd_v[sl] = sd_v[sl] + rbase
                return carry
            lax.fori_loop(0, C // 16, body, 0)

        def edge_chunks(r, base, rbase):
            def body(i, carry):
                j = s + NT * i
                off = pl.multiple_of((r * nchunks + j) * 2 * C, 8)
                pltpu.sync_copy(sd_hbm.at[pl.ds(off, 2 * C)], sd_v)
                remap(base, rbase)
                pltpu.async_copy(
                    p_hbm.at[sd_v.at[pl.ds(0, C)]], rows_v, sem).wait()
                pltpu.sync_copy(rows_v.at[pl.ds(0, 128)],
                                acc.at[dstl_v.at[0]], add=True)
                pltpu.sync_copy(rows_v.at[pl.ds(128, 128)],
                                acc.at[dstl_v.at[1]], add=True)
                return carry
            lax.fori_loop(0, nhi, body, 0)

        def drain(r, u0):
            @pl.when(s < 15)
            def _():
                pltpu.sync_copy(acc.at[pl.ds(row0, 632)],
                                s_hbm.at[r, pl.ds(u0 + row0, 632)])

            @pl.when(s == 15)
            def _():
                pltpu.sync_copy(acc.at[pl.ds(9480, 536)],
                                s_hbm.at[r, pl.ds(u0 + 9480, 536)])

        for gg in range(5):
            base = (c * 5 + gg) * RNG
            for r in range(R):
                zero_acc()
                plsc.subcore_barrier()
                edge_chunks(r, base, r * nitems)
                plsc.subcore_barrier()
                drain(r, base)

    return k_sc(p, sd)


def kernel(item_features, user_features, item_nids,
           edge_src_0, edge_dst_0, edge_src_1, edge_dst_1,
           edge_src_2, edge_dst_2, edge_src_3, edge_dst_3,
           edge_src_4, edge_dst_4,
           item_id_table,
           W_rev_0, b_rev_0, W_rev_1, b_rev_1, W_rev_2, b_rev_2,
           W_rev_3, b_rev_3, W_rev_4, b_rev_4,
           W_agg, b_agg):
    n_users = user_features.shape[0]
    e = edge_src_0.shape[0]
    item_id_emb = jnp.take(item_id_table, item_nids, axis=0)
    x = jnp.concatenate([item_features, item_id_emb], axis=1)
    wrev = jnp.stack([W_rev_0, W_rev_1, W_rev_2, W_rev_3, W_rev_4])
    brev = jnp.stack([b_rev_0, b_rev_1, b_rev_2, b_rev_3, b_rev_4])

    p = _item_transform(x, wrev, brev, W_agg)

    spad = jnp.zeros((EPAD - e,), jnp.int32)
    dpad = jnp.full((EPAD - e,), 1 << 20, jnp.int32)
    src1 = jnp.concatenate([
        edge_src_0.astype(jnp.int32), spad, edge_src_1.astype(jnp.int32), spad,
        edge_src_2.astype(jnp.int32), spad, edge_src_3.astype(jnp.int32), spad,
        edge_src_4.astype(jnp.int32), spad])
    dst1 = jnp.concatenate([
        edge_dst_0.astype(jnp.int32), dpad, edge_dst_1.astype(jnp.int32), dpad,
        edge_dst_2.astype(jnp.int32), dpad, edge_dst_3.astype(jnp.int32), dpad,
        edge_dst_4.astype(jnp.int32), dpad])
    sd = jnp.stack([src1.reshape(-1, C), dst1.reshape(-1, C)],
                   axis=1).reshape(-1)

    s = _edge_stage(p, sd, n_users)

    return _final(user_features, s, W_agg, b_agg.reshape(1, D))


# speedup vs baseline: 1.5389x; 1.1458x over previous
import functools
import jax
import jax.numpy as jnp
from jax import lax
from jax.experimental import pallas as pl
from jax.experimental.pallas import tpu as pltpu
from jax.experimental.pallas import tpu_sc as plsc

R = 5
DIN = 128
D = 64
C = 128
EPAD = 200064
RNG = 12512
ACCR = 12544
NT = 16


def _transform_body(x_ref, wrev_ref, brev_ref, wagg_ref, out_ref):
    x = x_ref[...]
    blk = x.shape[0]
    for r in range(R):
        m = jnp.dot(x, wrev_ref[r], preferred_element_type=jnp.float32) + brev_ref[r]
        p = jnp.dot(m, wagg_ref[pl.ds(D * (r + 1), D), :],
                    preferred_element_type=jnp.float32)
        out_ref[r] = jnp.concatenate(
            [p, jnp.ones((blk, 1), jnp.float32),
             jnp.zeros((blk, DIN - D - 1), jnp.float32)], axis=1)


def _item_transform(x, wrev, brev, wagg, block=2000):
    n = x.shape[0]
    return pl.pallas_call(
        _transform_body,
        grid=(n // block,),
        in_specs=[
            pl.BlockSpec((block, DIN), lambda i: (i, 0)),
            pl.BlockSpec((R, DIN, D), lambda i: (0, 0, 0)),
            pl.BlockSpec((R, D), lambda i: (0, 0)),
            pl.BlockSpec((D * (R + 1), D), lambda i: (0, 0)),
        ],
        out_specs=pl.BlockSpec((R, block, DIN), lambda i: (0, i, 0)),
        out_shape=jax.ShapeDtypeStruct((R, n, DIN), jnp.float32),
    )(x, wrev, brev, wagg)


def _final_body(uf_ref, s_ref, wagg_ref, bagg_ref, out_ref):
    acc = jnp.dot(uf_ref[...], wagg_ref[pl.ds(0, D), :],
                  preferred_element_type=jnp.float32)
    lanes = lax.broadcasted_iota(jnp.int32, (1, DIN), 1)
    cnt_hot = jnp.where(lanes == D, 1.0, 0.0)
    for r in range(R):
        srow = s_ref[r]
        cnt = jnp.sum(srow * cnt_hot, axis=1, keepdims=True)
        inv = 1.0 / jnp.maximum(cnt, 1.0)
        acc = acc + srow[:, :D] * inv
    acc = acc + bagg_ref[...]
    out_ref[...] = jnp.where(acc >= 0, acc, 0.01 * acc)


def _final(uf, s, wagg, bagg, block=2000):
    u = uf.shape[0]
    return pl.pallas_call(
        _final_body,
        grid=(u // block,),
        in_specs=[
            pl.BlockSpec((block, D), lambda i: (i, 0)),
            pl.BlockSpec((R, block, DIN), lambda i: (0, i, 0)),
            pl.BlockSpec((D * (R + 1), D), lambda i: (0, 0)),
            pl.BlockSpec((1, D), lambda i: (0, 0)),
        ],
        out_specs=pl.BlockSpec((block, D), lambda i: (i, 0)),
        out_shape=jax.ShapeDtypeStruct((u, D), jnp.float32),
    )(uf, s, wagg, bagg)


SOUT = 8 * RNG


def _edge_stage(p, sd, n_users):
    nitems = p.shape[1]
    p = p.reshape(R * nitems, DIN)
    nchunks = EPAD // C
    iters_long = nchunks - NT * (nchunks // NT)
    mesh = plsc.VectorSubcoreMesh(core_axis_name="c", subcore_axis_name="s")

    @functools.partial(
        pl.kernel, mesh=mesh,
        out_type=jax.ShapeDtypeStruct((R, SOUT, DIN), jnp.float32),
        scratch_types=[
            pltpu.VMEM_SHARED((ACCR, DIN), jnp.float32),
            pltpu.VMEM((2 * C,), jnp.int32),
            pltpu.VMEM((C,), jnp.int32),
            pltpu.VMEM((C, DIN), jnp.float32),
            pltpu.SemaphoreType.DMA,
        ],
    )
    def k_sc(p_hbm, sd_hbm, s_hbm, acc, sd_v, dstl_v, rows_v, sem):
        c = lax.axis_index("c")
        s = lax.axis_index("s")
        zvec = jnp.zeros((16,), jnp.float32)
        nhi = jnp.where(s < iters_long, nchunks // NT + 1, nchunks // NT)
        row0 = s * 784

        def zero_acc():
            def zfill(i, carry):
                for k in range(DIN // 16):
                    rows_v[i, pl.ds(k * 16, 16)] = zvec
                return carry
            lax.fori_loop(0, C, zfill, 0)
            for k in range(6):
                pltpu.sync_copy(rows_v, acc.at[pl.ds(row0 + C * k, C)])
            pltpu.sync_copy(rows_v.at[pl.ds(0, 16)],
                            acc.at[pl.ds(row0 + 6 * C, 16)])

        def remap(base, rbase):
            def body(k, carry):
                sl = pl.ds(k * 16, 16)
                loc = sd_v[pl.ds(C + k * 16, 16)] - base
                ok = (loc >= 0) & (loc < RNG)
                dstl_v[sl] = jnp.where(ok, loc, RNG)
                sd_v[sl] = sd_v[sl] + rbase
                return carry
            lax.fori_loop(0, C // 16, body, 0)

        def edge_chunks(r, base, rbase):
            def body(i, carry):
                j = s + NT * i
                off = pl.multiple_of((r * nchunks + j) * 2 * C, 8)
                pltpu.sync_copy(sd_hbm.at[pl.ds(off, 2 * C)], sd_v)
                remap(base, rbase)
                pltpu.async_copy(
                    p_hbm.at[sd_v.at[pl.ds(0, C)]], rows_v, sem).wait()
                pltpu.sync_copy(rows_v, acc.at[dstl_v], add=True)
                return carry
            lax.fori_loop(0, nhi, body, 0)

        def drain(r, u0):
            @pl.when(s < 15)
            def _():
                pltpu.sync_copy(acc.at[pl.ds(row0, 784)],
                                s_hbm.at[r, pl.ds(u0 + row0, 784)])

            @pl.when(s == 15)
            def _():
                pltpu.sync_copy(acc.at[pl.ds(11760, 752)],
                                s_hbm.at[r, pl.ds(u0 + 11760, 752)])

        for gg in range(4):
            base = (c * 4 + gg) * RNG
            for r in range(R):
                zero_acc()
                plsc.subcore_barrier()
                edge_chunks(r, base, r * nitems)
                plsc.subcore_barrier()
                drain(r, base)

    return k_sc(p, sd)


def kernel(item_features, user_features, item_nids,
           edge_src_0, edge_dst_0, edge_src_1, edge_dst_1,
           edge_src_2, edge_dst_2, edge_src_3, edge_dst_3,
           edge_src_4, edge_dst_4,
           item_id_table,
           W_rev_0, b_rev_0, W_rev_1, b_rev_1, W_rev_2, b_rev_2,
           W_rev_3, b_rev_3, W_rev_4, b_rev_4,
           W_agg, b_agg):
    n_users = user_features.shape[0]
    e = edge_src_0.shape[0]
    item_id_emb = jnp.take(item_id_table, item_nids, axis=0)
    x = jnp.concatenate([item_features, item_id_emb], axis=1)
    wrev = jnp.stack([W_rev_0, W_rev_1, W_rev_2, W_rev_3, W_rev_4])
    brev = jnp.stack([b_rev_0, b_rev_1, b_rev_2, b_rev_3, b_rev_4])

    p = _item_transform(x, wrev, brev, W_agg)

    spad = jnp.zeros((EPAD - e,), jnp.int32)
    dpad = jnp.full((EPAD - e,), 1 << 20, jnp.int32)
    src1 = jnp.concatenate([
        edge_src_0.astype(jnp.int32), spad, edge_src_1.astype(jnp.int32), spad,
        edge_src_2.astype(jnp.int32), spad, edge_src_3.astype(jnp.int32), spad,
        edge_src_4.astype(jnp.int32), spad])
    dst1 = jnp.concatenate([
        edge_dst_0.astype(jnp.int32), dpad, edge_dst_1.astype(jnp.int32), dpad,
        edge_dst_2.astype(jnp.int32), dpad, edge_dst_3.astype(jnp.int32), dpad,
        edge_dst_4.astype(jnp.int32), dpad])
    sd = jnp.stack([src1.reshape(-1, C), dst1.reshape(-1, C)],
                   axis=1).reshape(-1)

    s = _edge_stage(p, sd, n_users)

    return _final(user_features, s, W_agg, b_agg.reshape(1, D))


# speedup vs baseline: 1.7416x; 1.1317x over previous
import functools
import jax
import jax.numpy as jnp
from jax import lax
from jax.experimental import pallas as pl
from jax.experimental.pallas import tpu as pltpu
from jax.experimental.pallas import tpu_sc as plsc

R = 5
DIN = 128
D = 64
C = 128
EPAD = 200064
RNG = 12512
ACCR = 12544
NT = 16


def _transform_body(x_ref, wrev_ref, brev_ref, wagg_ref, out_ref):
    x = x_ref[...]
    blk = x.shape[0]
    for r in range(R):
        m = jnp.dot(x, wrev_ref[r], preferred_element_type=jnp.float32) + brev_ref[r]
        p = jnp.dot(m, wagg_ref[pl.ds(D * (r + 1), D), :],
                    preferred_element_type=jnp.float32)
        out_ref[r] = jnp.concatenate(
            [p, jnp.ones((blk, 1), jnp.float32),
             jnp.zeros((blk, DIN - D - 1), jnp.float32)], axis=1)


def _item_transform(x, wrev, brev, wagg, block=2000):
    n = x.shape[0]
    return pl.pallas_call(
        _transform_body,
        grid=(n // block,),
        in_specs=[
            pl.BlockSpec((block, DIN), lambda i: (i, 0)),
            pl.BlockSpec((R, DIN, D), lambda i: (0, 0, 0)),
            pl.BlockSpec((R, D), lambda i: (0, 0)),
            pl.BlockSpec((D * (R + 1), D), lambda i: (0, 0)),
        ],
        out_specs=pl.BlockSpec((R, block, DIN), lambda i: (0, i, 0)),
        out_shape=jax.ShapeDtypeStruct((R, n, DIN), jnp.float32),
    )(x, wrev, brev, wagg)


def _final_body(uf_ref, s_ref, wagg_ref, bagg_ref, out_ref):
    acc = jnp.dot(uf_ref[...], wagg_ref[pl.ds(0, D), :],
                  preferred_element_type=jnp.float32)
    lanes = lax.broadcasted_iota(jnp.int32, (1, DIN), 1)
    cnt_hot = jnp.where(lanes == D, 1.0, 0.0)
    for r in range(R):
        srow = s_ref[r]
        cnt = jnp.sum(srow * cnt_hot, axis=1, keepdims=True)
        inv = 1.0 / jnp.maximum(cnt, 1.0)
        acc = acc + srow[:, :D] * inv
    acc = acc + bagg_ref[...]
    out_ref[...] = jnp.where(acc >= 0, acc, 0.01 * acc)


def _final(uf, s, wagg, bagg, block=2000):
    u = uf.shape[0]
    return pl.pallas_call(
        _final_body,
        grid=(u // block,),
        in_specs=[
            pl.BlockSpec((block, D), lambda i: (i, 0)),
            pl.BlockSpec((R, block, DIN), lambda i: (0, i, 0)),
            pl.BlockSpec((D * (R + 1), D), lambda i: (0, 0)),
            pl.BlockSpec((1, D), lambda i: (0, 0)),
        ],
        out_specs=pl.BlockSpec((block, D), lambda i: (i, 0)),
        out_shape=jax.ShapeDtypeStruct((u, D), jnp.float32),
    )(uf, s, wagg, bagg)


SOUT = 8 * RNG


def _edge_stage(p, sd, n_users):
    nitems = p.shape[1]
    p = p.reshape(R * nitems, DIN)
    nchunks = EPAD // C
    iters_long = nchunks - NT * (nchunks // NT)
    mesh = plsc.VectorSubcoreMesh(core_axis_name="c", subcore_axis_name="s")

    @functools.partial(
        pl.kernel, mesh=mesh,
        out_type=jax.ShapeDtypeStruct((R, SOUT, DIN), jnp.float32),
        scratch_types=[
            pltpu.VMEM_SHARED((ACCR, DIN), jnp.float32),
            pltpu.VMEM((2 * C,), jnp.int32),
            pltpu.VMEM((2 * C,), jnp.int32),
            pltpu.VMEM((C,), jnp.int32),
            pltpu.VMEM((C,), jnp.int32),
            pltpu.VMEM((C, DIN), jnp.float32),
            pltpu.SemaphoreType.DMA,
            pltpu.SemaphoreType.DMA,
            pltpu.SemaphoreType.DMA,
        ],
    )
    def k_sc(p_hbm, sd_hbm, s_hbm, acc, sd_v0, sd_v1, srcg_v, dstl_v,
             rows_v, sem, sem_i0, sem_i1):
        c = lax.axis_index("c")
        s = lax.axis_index("s")
        zvec = jnp.zeros((16,), jnp.float32)
        nhi = jnp.where(s < iters_long, nchunks // NT + 1, nchunks // NT)
        row0 = s * 784

        def zero_acc():
            def zfill(i, carry):
                for k in range(DIN // 16):
                    rows_v[i, pl.ds(k * 16, 16)] = zvec
                return carry
            lax.fori_loop(0, C, zfill, 0)
            for k in range(6):
                pltpu.sync_copy(rows_v, acc.at[pl.ds(row0 + C * k, C)])
            pltpu.sync_copy(rows_v.at[pl.ds(0, 16)],
                            acc.at[pl.ds(row0 + 6 * C, 16)])

        sds = (sd_v0, sd_v1)
        semi = (sem_i0, sem_i1)

        def remap(sdref, base, rbase):
            def body(k, carry):
                sl = pl.ds(k * 16, 16)
                loc = sdref[pl.ds(C + k * 16, 16)] - base
                ok = (loc >= 0) & (loc < RNG)
                dstl_v[sl] = jnp.where(ok, loc, RNG)
                srcg_v[sl] = sdref[sl] + rbase
                return carry
            lax.fori_loop(0, C // 16, body, 0)

        def issue_sd(b, r, i):
            j = s + NT * i
            off = pl.multiple_of((r * nchunks + j) * 2 * C, 8)
            pltpu.async_copy(sd_hbm.at[pl.ds(off, 2 * C)], sds[b], semi[b])

        def wait_sd(b):
            pltpu.make_async_copy(sd_hbm.at[pl.ds(0, 2 * C)], sds[b],
                                  semi[b]).wait()

        def edge_chunks(r, base, rbase):
            issue_sd(0, r, 0)

            def body(i, carry):
                par = lax.rem(i, 2)
                for b in (0, 1):
                    @pl.when(par == b)
                    def _(b=b):
                        wait_sd(b)

                        @pl.when(i < nhi - 1)
                        def _():
                            issue_sd(1 - b, r, i + 1)
                        remap(sds[b], base, rbase)
                pltpu.async_copy(p_hbm.at[srcg_v], rows_v, sem).wait()
                pltpu.sync_copy(rows_v, acc.at[dstl_v], add=True)
                return carry
            lax.fori_loop(0, nhi, body, 0)

        def drain(r, u0):
            @pl.when(s < 15)
            def _():
                pltpu.sync_copy(acc.at[pl.ds(row0, 784)],
                                s_hbm.at[r, pl.ds(u0 + row0, 784)])

            @pl.when(s == 15)
            def _():
                pltpu.sync_copy(acc.at[pl.ds(11760, 752)],
                                s_hbm.at[r, pl.ds(u0 + 11760, 752)])

        for gg in range(4):
            base = (c * 4 + gg) * RNG
            for r in range(R):
                zero_acc()
                plsc.subcore_barrier()
                edge_chunks(r, base, r * nitems)
                plsc.subcore_barrier()
                drain(r, base)

    return k_sc(p, sd)


def kernel(item_features, user_features, item_nids,
           edge_src_0, edge_dst_0, edge_src_1, edge_dst_1,
           edge_src_2, edge_dst_2, edge_src_3, edge_dst_3,
           edge_src_4, edge_dst_4,
           item_id_table,
           W_rev_0, b_rev_0, W_rev_1, b_rev_1, W_rev_2, b_rev_2,
           W_rev_3, b_rev_3, W_rev_4, b_rev_4,
           W_agg, b_agg):
    n_users = user_features.shape[0]
    e = edge_src_0.shape[0]
    item_id_emb = jnp.take(item_id_table, item_nids, axis=0)
    x = jnp.concatenate([item_features, item_id_emb], axis=1)
    wrev = jnp.stack([W_rev_0, W_rev_1, W_rev_2, W_rev_3, W_rev_4])
    brev = jnp.stack([b_rev_0, b_rev_1, b_rev_2, b_rev_3, b_rev_4])

    p = _item_transform(x, wrev, brev, W_agg)

    spad = jnp.zeros((EPAD - e,), jnp.int32)
    dpad = jnp.full((EPAD - e,), 1 << 20, jnp.int32)
    src1 = jnp.concatenate([
        edge_src_0.astype(jnp.int32), spad, edge_src_1.astype(jnp.int32), spad,
        edge_src_2.astype(jnp.int32), spad, edge_src_3.astype(jnp.int32), spad,
        edge_src_4.astype(jnp.int32), spad])
    dst1 = jnp.concatenate([
        edge_dst_0.astype(jnp.int32), dpad, edge_dst_1.astype(jnp.int32), dpad,
        edge_dst_2.astype(jnp.int32), dpad, edge_dst_3.astype(jnp.int32), dpad,
        edge_dst_4.astype(jnp.int32), dpad])
    sd = jnp.stack([src1.reshape(-1, C), dst1.reshape(-1, C)],
                   axis=1).reshape(-1)

    s = _edge_stage(p, sd, n_users)

    return _final(user_features, s, W_agg, b_agg.reshape(1, D))


# speedup vs baseline: 1.7486x; 1.0040x over previous
import functools
import jax
import jax.numpy as jnp
from jax import lax
from jax.experimental import pallas as pl
from jax.experimental.pallas import tpu as pltpu
from jax.experimental.pallas import tpu_sc as plsc

R = 5
DIN = 128
D = 64
C = 128
EPAD = 200064
RNG = 12512
ACCR = 12544
NT = 16


def _transform_body(x_ref, wrev_ref, brev_ref, wagg_ref, out_ref):
    x = x_ref[...]
    blk = x.shape[0]
    for r in range(R):
        m = jnp.dot(x, wrev_ref[r], preferred_element_type=jnp.float32) + brev_ref[r]
        p = jnp.dot(m, wagg_ref[pl.ds(D * (r + 1), D), :],
                    preferred_element_type=jnp.float32)
        out_ref[r] = jnp.concatenate(
            [p, jnp.ones((blk, 1), jnp.float32),
             jnp.zeros((blk, DIN - D - 1), jnp.float32)], axis=1)


def _item_transform(x, wrev, brev, wagg, block=2000):
    n = x.shape[0]
    return pl.pallas_call(
        _transform_body,
        grid=(n // block,),
        in_specs=[
            pl.BlockSpec((block, DIN), lambda i: (i, 0)),
            pl.BlockSpec((R, DIN, D), lambda i: (0, 0, 0)),
            pl.BlockSpec((R, D), lambda i: (0, 0)),
            pl.BlockSpec((D * (R + 1), D), lambda i: (0, 0)),
        ],
        out_specs=pl.BlockSpec((R, block, DIN), lambda i: (0, i, 0)),
        out_shape=jax.ShapeDtypeStruct((R, n, DIN), jnp.float32),
    )(x, wrev, brev, wagg)


def _final_body(uf_ref, s_ref, wagg_ref, bagg_ref, out_ref):
    acc = jnp.dot(uf_ref[...], wagg_ref[pl.ds(0, D), :],
                  preferred_element_type=jnp.float32)
    lanes = lax.broadcasted_iota(jnp.int32, (1, DIN), 1)
    cnt_hot = jnp.where(lanes == D, 1.0, 0.0)
    for r in range(R):
        srow = s_ref[r]
        cnt = jnp.sum(srow * cnt_hot, axis=1, keepdims=True)
        inv = 1.0 / jnp.maximum(cnt, 1.0)
        acc = acc + srow[:, :D] * inv
    acc = acc + bagg_ref[...]
    out_ref[...] = jnp.where(acc >= 0, acc, 0.01 * acc)


def _final(uf, s, wagg, bagg, block=2000):
    u = uf.shape[0]
    return pl.pallas_call(
        _final_body,
        grid=(u // block,),
        in_specs=[
            pl.BlockSpec((block, D), lambda i: (i, 0)),
            pl.BlockSpec((R, block, DIN), lambda i: (0, i, 0)),
            pl.BlockSpec((D * (R + 1), D), lambda i: (0, 0)),
            pl.BlockSpec((1, D), lambda i: (0, 0)),
        ],
        out_specs=pl.BlockSpec((block, D), lambda i: (i, 0)),
        out_shape=jax.ShapeDtypeStruct((u, D), jnp.float32),
    )(uf, s, wagg, bagg)


SOUT = 8 * RNG


def _edge_stage(p, sd, n_users):
    nitems = p.shape[1]
    p = p.reshape(R * nitems, DIN)
    nchunks = EPAD // C
    iters_long = nchunks - NT * (nchunks // NT)
    mesh = plsc.VectorSubcoreMesh(core_axis_name="c", subcore_axis_name="s")

    @functools.partial(
        pl.kernel, mesh=mesh,
        out_type=jax.ShapeDtypeStruct((R, SOUT, DIN), jnp.float32),
        scratch_types=[
            pltpu.VMEM_SHARED((ACCR, DIN), jnp.float32),
            pltpu.VMEM((2 * C,), jnp.int32),
            pltpu.VMEM((2 * C,), jnp.int32),
            pltpu.VMEM((C,), jnp.int32),
            pltpu.VMEM((C,), jnp.int32),
            pltpu.VMEM((C,), jnp.int32),
            pltpu.VMEM((C, DIN), jnp.float32),
            pltpu.SemaphoreType.DMA,
            pltpu.SemaphoreType.DMA,
            pltpu.SemaphoreType.DMA,
            pltpu.SemaphoreType.DMA,
        ],
    )
    def k_sc(p_hbm, sd_hbm, s_hbm, acc, sd_v0, sd_v1, srcg_v,
             dstl_v0, dstl_v1, rows_v, sem, sem_i0, sem_i1, sem_s):
        c = lax.axis_index("c")
        s = lax.axis_index("s")
        zvec = jnp.zeros((16,), jnp.float32)
        nhi = jnp.where(s < iters_long, nchunks // NT + 1, nchunks // NT)
        row0 = s * 784

        def zero_acc():
            def zfill(i, carry):
                for k in range(DIN // 16):
                    rows_v[i, pl.ds(k * 16, 16)] = zvec
                return carry
            lax.fori_loop(0, C, zfill, 0)
            for k in range(6):
                pltpu.sync_copy(rows_v, acc.at[pl.ds(row0 + C * k, C)])
            pltpu.sync_copy(rows_v.at[pl.ds(0, 16)],
                            acc.at[pl.ds(row0 + 6 * C, 16)])

        sds = (sd_v0, sd_v1)
        dstls = (dstl_v0, dstl_v1)
        semi = (sem_i0, sem_i1)

        def remap(sdref, dstlref, base, rbase):
            def body(k, carry):
                sl = pl.ds(k * 16, 16)
                loc = sdref[pl.ds(C + k * 16, 16)] - base
                ok = (loc >= 0) & (loc < RNG)
                dstlref[sl] = jnp.where(ok, loc, RNG)
                srcg_v[sl] = sdref[sl] + rbase
                return carry
            lax.fori_loop(0, C // 16, body, 0)

        def issue_sd(b, r, i):
            j = s + NT * i
            off = pl.multiple_of((r * nchunks + j) * 2 * C, 8)
            pltpu.async_copy(sd_hbm.at[pl.ds(off, 2 * C)], sds[b], semi[b])

        def wait_sd(b):
            pltpu.make_async_copy(sd_hbm.at[pl.ds(0, 2 * C)], sds[b],
                                  semi[b]).wait()

        def wait_scatter(b):
            pltpu.make_async_copy(rows_v, acc.at[dstls[b]], sem_s).wait()

        def edge_chunks(r, base, rbase):
            issue_sd(0, r, 0)

            def body(i, carry):
                par = lax.rem(i, 2)
                for b in (0, 1):
                    @pl.when(par == b)
                    def _(b=b):
                        wait_sd(b)

                        @pl.when(i < nhi - 1)
                        def _():
                            issue_sd(1 - b, r, i + 1)
                        remap(sds[b], dstls[b], base, rbase)

                        @pl.when(i > 0)
                        def _():
                            wait_scatter(1 - b)
                        pltpu.async_copy(p_hbm.at[srcg_v], rows_v, sem).wait()
                        pltpu.async_copy(rows_v, acc.at[dstls[b]], sem_s,
                                         add=True)
                return carry
            lax.fori_loop(0, nhi, body, 0)
            wait_scatter(0)

        def drain(r, u0):
            @pl.when(s < 15)
            def _():
                pltpu.sync_copy(acc.at[pl.ds(row0, 784)],
                                s_hbm.at[r, pl.ds(u0 + row0, 784)])

            @pl.when(s == 15)
            def _():
                pltpu.sync_copy(acc.at[pl.ds(11760, 752)],
                                s_hbm.at[r, pl.ds(u0 + 11760, 752)])

        for gg in range(4):
            base = (c * 4 + gg) * RNG
            for r in range(R):
                zero_acc()
                plsc.subcore_barrier()
                edge_chunks(r, base, r * nitems)
                plsc.subcore_barrier()
                drain(r, base)

    return k_sc(p, sd)


def kernel(item_features, user_features, item_nids,
           edge_src_0, edge_dst_0, edge_src_1, edge_dst_1,
           edge_src_2, edge_dst_2, edge_src_3, edge_dst_3,
           edge_src_4, edge_dst_4,
           item_id_table,
           W_rev_0, b_rev_0, W_rev_1, b_rev_1, W_rev_2, b_rev_2,
           W_rev_3, b_rev_3, W_rev_4, b_rev_4,
           W_agg, b_agg):
    n_users = user_features.shape[0]
    e = edge_src_0.shape[0]
    item_id_emb = jnp.take(item_id_table, item_nids, axis=0)
    x = jnp.concatenate([item_features, item_id_emb], axis=1)
    wrev = jnp.stack([W_rev_0, W_rev_1, W_rev_2, W_rev_3, W_rev_4])
    brev = jnp.stack([b_rev_0, b_rev_1, b_rev_2, b_rev_3, b_rev_4])

    p = _item_transform(x, wrev, brev, W_agg)

    spad = jnp.zeros((EPAD - e,), jnp.int32)
    dpad = jnp.full((EPAD - e,), 1 << 20, jnp.int32)
    src1 = jnp.concatenate([
        edge_src_0.astype(jnp.int32), spad, edge_src_1.astype(jnp.int32), spad,
        edge_src_2.astype(jnp.int32), spad, edge_src_3.astype(jnp.int32), spad,
        edge_src_4.astype(jnp.int32), spad])
    dst1 = jnp.concatenate([
        edge_dst_0.astype(jnp.int32), dpad, edge_dst_1.astype(jnp.int32), dpad,
        edge_dst_2.astype(jnp.int32), dpad, edge_dst_3.astype(jnp.int32), dpad,
        edge_dst_4.astype(jnp.int32), dpad])
    sd = jnp.stack([src1.reshape(-1, C), dst1.reshape(-1, C)],
                   axis=1).reshape(-1)

    s = _edge_stage(p, sd, n_users)

    return _final(user_features, s, W_agg, b_agg.reshape(1, D))
